# baseline (speedup 1.0000x reference)
import jax
import jax.numpy as jnp
from jax.experimental import pallas as pl
from jax.experimental.pallas import tpu as pltpu

_NDMA = 8


def _dma_body(in_ref, out_ref, *sems):
    rows = in_ref.shape[0]
    chunk = rows // _NDMA
    copies = []
    for i in range(_NDMA):
        c = pltpu.make_async_copy(
            in_ref.at[pl.ds(i * chunk, chunk)],
            out_ref.at[pl.ds(i * chunk, chunk)],
            sems[i],
        )
        c.start()
        copies.append(c)
    for c in copies:
        c.wait()


def kernel(x, pos_table):
    seq_len = x.shape[1]
    d_model = pos_table.shape[1]
    table = pos_table[:seq_len]
    out = pl.pallas_call(
        _dma_body,
        in_specs=[pl.BlockSpec(memory_space=pl.ANY)],
        out_specs=pl.BlockSpec(memory_space=pl.ANY),
        out_shape=jax.ShapeDtypeStruct((seq_len, d_model), pos_table.dtype),
        scratch_shapes=[pltpu.SemaphoreType.DMA] * _NDMA,
    )(table)
    return out[None]


# speedup vs baseline: 22.7735x; 22.7735x over previous
import functools

import jax
import jax.numpy as jnp
from jax import lax
from jax.experimental import pallas as pl
from jax.experimental.pallas import tpu as pltpu
from jax.experimental.pallas import tpu_sc as plsc

_CHUNK_ROWS = 32
_NBUF = 3


def _make_sc_copy(seq_len, d_model, dtype):
    info = plsc.get_sparse_core_info()
    nc, ns = info.num_cores, info.num_subcores
    nw = nc * ns
    rows_per = seq_len // nw
    nchunks = rows_per // _CHUNK_ROWS
    mesh = plsc.VectorSubcoreMesh(core_axis_name="c", subcore_axis_name="s")

    scratch = [pltpu.VMEM((_CHUNK_ROWS, d_model), dtype)] * _NBUF
    scratch += [pltpu.SemaphoreType.DMA] * (2 * _NBUF)

    @functools.partial(
        pl.kernel,
        mesh=mesh,
        out_type=jax.ShapeDtypeStruct((seq_len, d_model), dtype),
        scratch_types=scratch,
    )
    def sc_copy(table_hbm, out_hbm, *scr):
        bufs = scr[:_NBUF]
        gsem = scr[_NBUF:2 * _NBUF]
        ssem = scr[2 * _NBUF:]
        wid = lax.axis_index("c") * ns + lax.axis_index("s")
        base = wid * rows_per
        scat = [None] * _NBUF
        for i in range(nchunks):
            b = i % _NBUF
            lo = base + i * _CHUNK_ROWS
            if scat[b] is not None:
                scat[b].wait()
            gath = pltpu.async_copy(
                table_hbm.at[pl.ds(lo, _CHUNK_ROWS)], bufs[b], gsem[b]
            )
            gath.wait()
            scat[b] = pltpu.async_copy(
                bufs[b], out_hbm.at[pl.ds(lo, _CHUNK_ROWS)], ssem[b]
            )
        for b in range(_NBUF):
            if scat[b] is not None:
                scat[b].wait()

    return sc_copy


def kernel(x, pos_table):
    seq_len = x.shape[1]
    d_model = pos_table.shape[1]
    table = pos_table[:seq_len]
    out = _make_sc_copy(seq_len, d_model, pos_table.dtype)(table)
    return out[None]
